# baseline (speedup 1.0000x reference)
import functools

import jax
import jax.numpy as jnp
from jax.experimental import pallas as pl
from jax.experimental.pallas import tpu as pltpu

N_ROWS = 16384
N_CLS = 1000
BLK = 512
NBUF = 4
NSTEPS = N_ROWS // BLK


def _selflabel_body(anchor_hbm, aug_hbm, out_ref, abuf, gbuf, tvmem, asem, gsem):
    def a_copy(step, slot):
        return pltpu.make_async_copy(
            anchor_hbm.at[pl.ds(step * BLK, BLK), :], abuf.at[slot], asem.at[slot]
        )

    def g_copy(step, slot):
        return pltpu.make_async_copy(
            aug_hbm.at[pl.ds(step * BLK, BLK), :], gbuf.at[slot], gsem.at[slot]
        )

    col = jax.lax.broadcasted_iota(jnp.int32, (BLK, N_CLS), 1)

    for p in range(NBUF):
        a_copy(p, p).start()

    def phase_a(i, carry):
        slot = jax.lax.rem(i, NBUF)
        a_copy(i, slot).wait()
        a = abuf[slot]
        row_max = jnp.max(a, axis=1, keepdims=True)
        t = jnp.min(jnp.where(a == row_max, col, N_CLS), axis=1, keepdims=True)
        tvmem[pl.ds(i * BLK, BLK), :] = t

        @pl.when(i + NBUF < NSTEPS)
        def _prefetch():
            a_copy(i + NBUF, slot).start()

        return carry

    jax.lax.fori_loop(0, NSTEPS, phase_a, 0)

    for p in range(NBUF):
        g_copy(p, p).start()

    def phase_b(i, carry):
        cnt_acc, ns_acc = carry
        slot = jax.lax.rem(i, NBUF)
        g_copy(i, slot).wait()
        g = gbuf[slot]
        t = tvmem[pl.ds(i * BLK, BLK), :]

        g_max = jnp.max(g, axis=1, keepdims=True)
        ssum = jnp.sum(jnp.exp(g - g_max), axis=1, keepdims=True)
        lse = jnp.log(ssum) + g_max

        onehot = col == t
        g_t = jnp.sum(jnp.where(onehot, g, 0.0), axis=1, keepdims=True)
        nll = lse - g_t

        cnt_acc = cnt_acc + jnp.sum(onehot.astype(jnp.float32), axis=0)[None, :]
        ns_acc = ns_acc + jnp.sum(jnp.where(onehot, nll, 0.0), axis=0)[None, :]

        @pl.when(i + NBUF < NSTEPS)
        def _prefetch():
            g_copy(i + NBUF, slot).start()

        return cnt_acc, ns_acc

    zero = jnp.zeros((1, N_CLS), jnp.float32)
    counts, ns = jax.lax.fori_loop(0, NSTEPS, phase_b, (zero, zero))

    nz = counts > 0.0
    k = jnp.sum(nz.astype(jnp.float32), axis=1, keepdims=True)
    per_cls = jnp.where(nz, ns / jnp.where(nz, counts, 1.0), 0.0)
    out_ref[...] = jnp.sum(per_cls, axis=1, keepdims=True) / k


@functools.partial(jax.jit, static_argnames=("interpret",))
def kernel(anchor_logits, aug_logits, interpret=False):
    out = pl.pallas_call(
        _selflabel_body,
        in_specs=[
            pl.BlockSpec(memory_space=pltpu.MemorySpace.HBM),
            pl.BlockSpec(memory_space=pltpu.MemorySpace.HBM),
        ],
        out_specs=pl.BlockSpec(memory_space=pltpu.MemorySpace.VMEM),
        out_shape=jax.ShapeDtypeStruct((1, 1), jnp.float32),
        scratch_shapes=[
            pltpu.VMEM((NBUF, BLK, N_CLS), jnp.float32),
            pltpu.VMEM((NBUF, BLK, N_CLS), jnp.float32),
            pltpu.VMEM((N_ROWS, 1), jnp.int32),
            pltpu.SemaphoreType.DMA((NBUF,)),
            pltpu.SemaphoreType.DMA((NBUF,)),
        ],
        interpret=interpret,
    )(anchor_logits, aug_logits)
    return out[0, 0]


# speedup vs baseline: 1.0458x; 1.0458x over previous
import functools

import jax
import jax.numpy as jnp
from jax.experimental import pallas as pl
from jax.experimental.pallas import tpu as pltpu

N_ROWS = 16384
N_CLS = 1000
BLK = 512
NBUF = 4
NSTEPS = N_ROWS // BLK
IDX_BITS = 0x3FF


def _selflabel_body(anchor_hbm, aug_hbm, out_ref, abuf, gbuf, tvmem, asem, gsem):
    def a_copy(step, slot):
        return pltpu.make_async_copy(
            anchor_hbm.at[pl.ds(step * BLK, BLK), :], abuf.at[slot], asem.at[slot]
        )

    def g_copy(step, slot):
        return pltpu.make_async_copy(
            aug_hbm.at[pl.ds(step * BLK, BLK), :], gbuf.at[slot], gsem.at[slot]
        )

    col = jax.lax.broadcasted_iota(jnp.int32, (BLK, N_CLS), 1)
    rev_col = IDX_BITS - col

    for p in range(NBUF):
        a_copy(p, p).start()

    def phase_a(i, carry):
        slot = jax.lax.rem(i, NBUF)
        a_copy(i, slot).wait()
        a = abuf[slot]
        ai = jax.lax.bitcast_convert_type(a, jnp.int32)
        key = ai ^ jax.lax.shift_right_logical(
            jax.lax.shift_right_arithmetic(ai, 31), 1
        )
        packed = (key & ~IDX_BITS) | rev_col
        best = jnp.max(packed, axis=1, keepdims=True)
        t = IDX_BITS - (best & IDX_BITS)
        tvmem[pl.ds(i * BLK, BLK), :] = t

        @pl.when(i + NBUF < NSTEPS)
        def _prefetch():
            a_copy(i + NBUF, slot).start()

        return carry

    jax.lax.fori_loop(0, NSTEPS, phase_a, 0)

    for p in range(NBUF):
        g_copy(p, p).start()

    def phase_b(i, carry):
        cnt_acc, ns_acc = carry
        slot = jax.lax.rem(i, NBUF)
        g_copy(i, slot).wait()
        g = gbuf[slot]
        t = tvmem[pl.ds(i * BLK, BLK), :]

        onehot = col == t
        ssum = jnp.sum(jnp.exp(g), axis=1, keepdims=True)
        g_t = jnp.sum(jnp.where(onehot, g, 0.0), axis=1, keepdims=True)
        nll = jnp.log(ssum) - g_t

        cnt_acc = cnt_acc + jnp.sum(onehot.astype(jnp.float32), axis=0)[None, :]
        ns_acc = ns_acc + jnp.sum(jnp.where(onehot, nll, 0.0), axis=0)[None, :]

        @pl.when(i + NBUF < NSTEPS)
        def _prefetch():
            g_copy(i + NBUF, slot).start()

        return cnt_acc, ns_acc

    zero = jnp.zeros((1, N_CLS), jnp.float32)
    counts, ns = jax.lax.fori_loop(0, NSTEPS, phase_b, (zero, zero))

    nz = counts > 0.0
    k = jnp.sum(nz.astype(jnp.float32), axis=1, keepdims=True)
    per_cls = jnp.where(nz, ns / jnp.where(nz, counts, 1.0), 0.0)
    out_ref[...] = jnp.sum(per_cls, axis=1, keepdims=True) / k


@functools.partial(jax.jit, static_argnames=("interpret",))
def kernel(anchor_logits, aug_logits, interpret=False):
    out = pl.pallas_call(
        _selflabel_body,
        in_specs=[
            pl.BlockSpec(memory_space=pltpu.MemorySpace.HBM),
            pl.BlockSpec(memory_space=pltpu.MemorySpace.HBM),
        ],
        out_specs=pl.BlockSpec(memory_space=pltpu.MemorySpace.VMEM),
        out_shape=jax.ShapeDtypeStruct((1, 1), jnp.float32),
        scratch_shapes=[
            pltpu.VMEM((NBUF, BLK, N_CLS), jnp.float32),
            pltpu.VMEM((NBUF, BLK, N_CLS), jnp.float32),
            pltpu.VMEM((N_ROWS, 1), jnp.int32),
            pltpu.SemaphoreType.DMA((NBUF,)),
            pltpu.SemaphoreType.DMA((NBUF,)),
        ],
        interpret=interpret,
    )(anchor_logits, aug_logits)
    return out[0, 0]


# speedup vs baseline: 1.0669x; 1.0202x over previous
import functools

import jax
import jax.numpy as jnp
from jax.experimental import pallas as pl
from jax.experimental.pallas import tpu as pltpu

N_ROWS = 16384
N_CLS = 1000
BLK = 512
NBUF = 4
NSTEPS = N_ROWS // BLK


def _selflabel_body(anchor_hbm, aug_hbm, out_ref, abuf, gbuf, asem, gsem):
    def a_copy(step, slot):
        return pltpu.make_async_copy(
            anchor_hbm.at[pl.ds(step * BLK, BLK), :], abuf.at[slot], asem.at[slot]
        )

    def g_copy(step, slot):
        return pltpu.make_async_copy(
            aug_hbm.at[pl.ds(step * BLK, BLK), :], gbuf.at[slot], gsem.at[slot]
        )

    for p in range(NBUF):
        a_copy(p, p).start()
        g_copy(p, p).start()

    col = jax.lax.broadcasted_iota(jnp.int32, (BLK, N_CLS), 1)

    def step_fn(i, carry):
        cnt_acc, ns_acc = carry
        slot = jax.lax.rem(i, NBUF)
        a_copy(i, slot).wait()
        g_copy(i, slot).wait()
        a = abuf[slot]
        g = gbuf[slot]

        row_max = jnp.max(a, axis=1, keepdims=True)
        t = jnp.min(jnp.where(a == row_max, col, N_CLS), axis=1, keepdims=True)

        g_max = jnp.max(g, axis=1, keepdims=True)
        ssum = jnp.sum(jnp.exp(g - g_max), axis=1, keepdims=True)
        lse = jnp.log(ssum) + g_max

        onehot = col == t
        g_t = jnp.sum(jnp.where(onehot, g, 0.0), axis=1, keepdims=True)
        nll = lse - g_t

        cnt_acc = cnt_acc + jnp.sum(onehot.astype(jnp.float32), axis=0)[None, :]
        ns_acc = ns_acc + jnp.sum(jnp.where(onehot, nll, 0.0), axis=0)[None, :]

        @pl.when(i + NBUF < NSTEPS)
        def _prefetch():
            a_copy(i + NBUF, slot).start()
            g_copy(i + NBUF, slot).start()

        return cnt_acc, ns_acc

    zero = jnp.zeros((1, N_CLS), jnp.float32)
    counts, ns = jax.lax.fori_loop(0, NSTEPS, step_fn, (zero, zero))

    nz = counts > 0.0
    k = jnp.sum(nz.astype(jnp.float32), axis=1, keepdims=True)
    per_cls = jnp.where(nz, ns / jnp.where(nz, counts, 1.0), 0.0)
    out_ref[...] = jnp.sum(per_cls, axis=1, keepdims=True) / k


@functools.partial(jax.jit, static_argnames=("interpret",))
def kernel(anchor_logits, aug_logits, interpret=False):
    out = pl.pallas_call(
        _selflabel_body,
        in_specs=[
            pl.BlockSpec(memory_space=pltpu.MemorySpace.HBM),
            pl.BlockSpec(memory_space=pltpu.MemorySpace.HBM),
        ],
        out_specs=pl.BlockSpec(memory_space=pltpu.MemorySpace.VMEM),
        out_shape=jax.ShapeDtypeStruct((1, 1), jnp.float32),
        scratch_shapes=[
            pltpu.VMEM((NBUF, BLK, N_CLS), jnp.float32),
            pltpu.VMEM((NBUF, BLK, N_CLS), jnp.float32),
            pltpu.SemaphoreType.DMA((NBUF,)),
            pltpu.SemaphoreType.DMA((NBUF,)),
        ],
        interpret=interpret,
    )(anchor_logits, aug_logits)
    return out[0, 0]
